# Initial kernel scaffold; baseline (speedup 1.0000x reference)
import jax
import jax.numpy as jnp
from jax.experimental import pallas as pl


def kernel(x, edge_index, W1, b1, W2, b2):
    raise NotImplementedError("write your pallas kernel here")


# speedup vs baseline: 11.3979x; 11.3979x over previous
import functools

import jax
import jax.numpy as jnp
from jax import lax
from jax.experimental import pallas as pl
from jax.experimental.pallas import tpu as pltpu
from jax.experimental.pallas import tpu_sc as plsc

N = 10000
E = 320000
DH = 128
DO = 8
DOP = 16
NC = 2
NS = 16
NW = NC * NS
EW = E // NW
K = 80
NCHUNK = EW // K
RQ = 624
TAIL = N - NS * RQ
NH = N // 2
NHP = NH + 8
GB = NH
RH = 312
HTAIL = NHP - NS * RH

_MESH = dict(core_axis_name="c", subcore_axis_name="s")


NR = 80


def _sc_degree(dst_r, eye_h, zeros_h):
    @functools.partial(
        pl.kernel,
        out_type=jax.ShapeDtypeStruct((NC, NR, DH), jnp.float32),
        mesh=plsc.VectorSubcoreMesh(**_MESH),
        scratch_types=[
            pltpu.VMEM((NCHUNK, K), jnp.int32),
            pltpu.VMEM((NCHUNK, K), jnp.int32),
            pltpu.VMEM((K, DH), jnp.float32),
            pltpu.VMEM((K, DH), jnp.float32),
            pltpu.VMEM_SHARED((NR, DH), jnp.float32),
            pltpu.SemaphoreType.DMA,
            pltpu.SemaphoreType.DMA,
        ],
    )
    def deg_kernel(dst_hbm, eye_hbm, z_hbm, out_hbm,
                   idx_g, idx_r, buf0, buf1, acc, sem0, sem1):
        c = lax.axis_index("c")
        s = lax.axis_index("s")
        wid = c * NS + s

        @pl.when(s < NR // 8)
        def _():
            pltpu.sync_copy(z_hbm.at[pl.ds(s * 8, 8)], acc.at[pl.ds(s * 8, 8)])

        pltpu.sync_copy(dst_hbm.at[wid], idx_r)

        @pl.loop(0, NCHUNK)
        def _(i):
            for k in range(K // 16):
                v = idx_r[i, pl.ds(k * 16, 16)]
                r = lax.shift_right_logical(v, 7)
                idx_g[i, pl.ds(k * 16, 16)] = v - r * 128
                idx_r[i, pl.ds(k * 16, 16)] = r

        plsc.subcore_barrier()

        pltpu.async_copy(eye_hbm.at[idx_g.at[0]], buf0, sem0)

        @pl.loop(0, NCHUNK - 1, step=2)
        def _(i):
            pltpu.async_copy(eye_hbm.at[idx_g.at[i + 1]], buf1, sem1)
            pltpu.make_async_copy(eye_hbm.at[idx_g.at[i]], buf0, sem0).wait()
            pltpu.sync_copy(buf0, acc.at[idx_r.at[i]], add=True)
            pltpu.async_copy(eye_hbm.at[idx_g.at[i + 2]], buf0, sem0)
            pltpu.make_async_copy(eye_hbm.at[idx_g.at[i + 1]], buf1, sem1).wait()
            pltpu.sync_copy(buf1, acc.at[idx_r.at[i + 1]], add=True)

        last = NCHUNK - 1
        pltpu.make_async_copy(eye_hbm.at[idx_g.at[last]], buf0, sem0).wait()
        pltpu.sync_copy(buf0, acc.at[idx_r.at[last]], add=True)

        plsc.subcore_barrier()

        @pl.when(s < NR // 8)
        def _():
            pltpu.sync_copy(acc.at[pl.ds(s * 8, 8)],
                            out_hbm.at[c, pl.ds(s * 8, 8)])

    return deg_kernel(dst_r, eye_h, zeros_h)


def _sc_aggregate(src_r, dst_r, rows, zeros_h, base):
    @functools.partial(
        pl.kernel,
        out_type=jax.ShapeDtypeStruct((NC, NHP, DH), jnp.float32),
        mesh=plsc.VectorSubcoreMesh(**_MESH),
        scratch_types=[
            pltpu.VMEM((NCHUNK, K), jnp.int32),
            pltpu.VMEM((NCHUNK, K), jnp.int32),
            pltpu.VMEM((K, DH), jnp.float32),
            pltpu.VMEM((K, DH), jnp.float32),
            pltpu.VMEM_SHARED((NHP, DH), jnp.float32),
            pltpu.SemaphoreType.DMA,
            pltpu.SemaphoreType.DMA,
        ],
    )
    def agg_kernel(src_hbm, dst_hbm, rows_hbm, z_hbm, out_hbm,
                   idx_s, idx_d, buf0, buf1, acc, sem0, sem1):
        c = lax.axis_index("c")
        s = lax.axis_index("s")
        wid = c * NS + s
        pltpu.sync_copy(z_hbm.at[pl.ds(s * RH, RH)], acc.at[pl.ds(s * RH, RH)])

        @pl.when(s == NS - 1)
        def _():
            pltpu.sync_copy(z_hbm.at[pl.ds(NS * RH, HTAIL)],
                            acc.at[pl.ds(NS * RH, HTAIL)])

        pltpu.sync_copy(src_hbm.at[wid], idx_s)
        pltpu.sync_copy(dst_hbm.at[wid], idx_d)

        @pl.loop(0, NCHUNK)
        def _(i):
            for k in range(K // 16):
                v = idx_d[i, pl.ds(k * 16, 16)]
                keep = jnp.logical_and(v >= base, v < base + NH)
                idx_d[i, pl.ds(k * 16, 16)] = jnp.where(keep, v - base, GB)

        plsc.subcore_barrier()

        pltpu.async_copy(rows_hbm.at[idx_s.at[0]], buf0, sem0)

        @pl.loop(0, NCHUNK - 1, step=2)
        def _(i):
            pltpu.async_copy(rows_hbm.at[idx_s.at[i + 1]], buf1, sem1)
            pltpu.make_async_copy(rows_hbm.at[idx_s.at[i]], buf0, sem0).wait()
            pltpu.sync_copy(buf0, acc.at[idx_d.at[i]], add=True)
            pltpu.async_copy(rows_hbm.at[idx_s.at[i + 2]], buf0, sem0)
            pltpu.make_async_copy(rows_hbm.at[idx_s.at[i + 1]], buf1, sem1).wait()
            pltpu.sync_copy(buf1, acc.at[idx_d.at[i + 1]], add=True)

        last = NCHUNK - 1
        pltpu.make_async_copy(rows_hbm.at[idx_s.at[last]], buf0, sem0).wait()
        pltpu.sync_copy(buf0, acc.at[idx_d.at[last]], add=True)

        plsc.subcore_barrier()
        pltpu.sync_copy(acc.at[pl.ds(s * RH, RH)],
                        out_hbm.at[c, pl.ds(s * RH, RH)])

        @pl.when(s == NS - 1)
        def _():
            pltpu.sync_copy(acc.at[pl.ds(NS * RH, HTAIL)],
                            out_hbm.at[c, pl.ds(NS * RH, HTAIL)])

    return agg_kernel(src_r, dst_r, rows, zeros_h)


def _tc_matmul(x, w):
    def body(x_ref, w_ref, o_ref):
        o_ref[...] = jnp.dot(x_ref[...], w_ref[...],
                             preferred_element_type=jnp.float32)

    return pl.pallas_call(
        body, out_shape=jax.ShapeDtypeStruct((x.shape[0], w.shape[1]),
                                             jnp.float32))(x, w)


def _tc_scale(xw, deg):
    def body(xw_ref, deg_ref, y_ref, dinv_ref):
        dinv = lax.rsqrt(deg_ref[...] + 1.0)
        dinv_ref[...] = dinv
        y_ref[...] = xw_ref[...] * dinv

    return pl.pallas_call(
        body,
        out_shape=[jax.ShapeDtypeStruct((N, DH), jnp.float32),
                   jax.ShapeDtypeStruct((N, 1), jnp.float32)])(xw, deg)


def _tc_mid(pa, pb, y1, dinv, b1):
    def body(pa_ref, pb_ref, y1_ref, dinv_ref, b1_ref, o_ref):
        agg_lo = pa_ref[0, 0:NH] + pa_ref[1, 0:NH]
        agg_hi = pb_ref[0, 0:NH] + pb_ref[1, 0:NH]
        agg = jnp.concatenate([agg_lo, agg_hi], axis=0)
        dv = dinv_ref[...]
        h = jnp.maximum((agg + y1_ref[...]) * dv + b1_ref[...], 0.0)
        o_ref[...] = h * dv

    return pl.pallas_call(
        body, out_shape=jax.ShapeDtypeStruct((N, DH), jnp.float32))(
            pa, pb, y1, dinv, b1)


def _tc_final(pa, pb, m, dinv, w2, b2):
    def body(pa_ref, pb_ref, m_ref, dinv_ref, w2_ref, b2_ref, o_ref):
        agg_lo = pa_ref[0, 0:NH] + pa_ref[1, 0:NH]
        agg_hi = pb_ref[0, 0:NH] + pb_ref[1, 0:NH]
        agg = jnp.concatenate([agg_lo, agg_hi], axis=0) + m_ref[...]
        o_ref[...] = (jnp.dot(agg, w2_ref[...],
                              preferred_element_type=jnp.float32)
                      * dinv_ref[...] + b2_ref[...])

    return pl.pallas_call(
        body, out_shape=jax.ShapeDtypeStruct((N, DO), jnp.float32))(
            pa, pb, m, dinv, w2, b2)


def kernel(x, edge_index, W1, b1, W2, b2):
    src_r = edge_index[0].reshape(NW, NCHUNK, K)
    dst_r = edge_index[1].reshape(NW, NCHUNK, K)
    zeros_h = jnp.zeros((NHP, DH), jnp.float32)
    eye128 = jnp.eye(DH, dtype=jnp.float32)
    b1r = b1.reshape(1, DH)
    b2r = b2.reshape(1, DO)

    degp = _sc_degree(dst_r, eye128, zeros_h)
    deg = (degp[0] + degp[1]).reshape(NR * DH)[:N].reshape(N, 1)
    xw = _tc_matmul(x, W1)
    y1, dinv = _tc_scale(xw, deg)
    p1a = _sc_aggregate(src_r, dst_r, y1, zeros_h, 0)
    p1b = _sc_aggregate(src_r, dst_r, y1, zeros_h, NH)
    m = _tc_mid(p1a, p1b, y1, dinv, b1r)
    p2a = _sc_aggregate(src_r, dst_r, m, zeros_h, 0)
    p2b = _sc_aggregate(src_r, dst_r, m, zeros_h, NH)
    return _tc_final(p2a, p2b, m, dinv, W2, b2r)


# speedup vs baseline: 14.2010x; 1.2459x over previous
import functools

import jax
import jax.numpy as jnp
from jax import lax
from jax.experimental import pallas as pl
from jax.experimental.pallas import tpu as pltpu
from jax.experimental.pallas import tpu_sc as plsc

N = 10000
E = 320000
DH = 128
DO = 8
DOP = 16
NC = 2
NS = 16
NW = NC * NS
EW = E // NW
K = 80
NCHUNK = EW // K
RQ = 624
TAIL = N - NS * RQ
NH = N // 2
NHP = NH + 8
GB = NH
RH = 312
HTAIL = NHP - NS * RH

_MESH = dict(core_axis_name="c", subcore_axis_name="s")


NR = 80


def _sc_degree(dst_r, eye_h, zeros_h):
    @functools.partial(
        pl.kernel,
        out_type=jax.ShapeDtypeStruct((NC, NR, DH), jnp.float32),
        mesh=plsc.VectorSubcoreMesh(**_MESH),
        scratch_types=[
            pltpu.VMEM((NCHUNK, K), jnp.int32),
            pltpu.VMEM((NCHUNK, K), jnp.int32),
            pltpu.VMEM((K, DH), jnp.float32),
            pltpu.VMEM((K, DH), jnp.float32),
            pltpu.VMEM_SHARED((NR, DH), jnp.float32),
            pltpu.SemaphoreType.DMA,
            pltpu.SemaphoreType.DMA,
        ],
    )
    def deg_kernel(dst_hbm, eye_hbm, z_hbm, out_hbm,
                   idx_g, idx_r, buf0, buf1, acc, sem0, sem1):
        c = lax.axis_index("c")
        s = lax.axis_index("s")
        wid = c * NS + s

        @pl.when(s < NR // 8)
        def _():
            pltpu.sync_copy(z_hbm.at[pl.ds(s * 8, 8)], acc.at[pl.ds(s * 8, 8)])

        pltpu.sync_copy(dst_hbm.at[wid], idx_r)

        @pl.loop(0, NCHUNK)
        def _(i):
            for k in range(K // 16):
                v = idx_r[i, pl.ds(k * 16, 16)]
                r = lax.shift_right_logical(v, 7)
                idx_g[i, pl.ds(k * 16, 16)] = (v - r * 128) + wid * DH
                idx_r[i, pl.ds(k * 16, 16)] = r

        plsc.subcore_barrier()

        pltpu.async_copy(eye_hbm.at[idx_g.at[0]], buf0, sem0)

        @pl.loop(0, NCHUNK - 1, step=2)
        def _(i):
            pltpu.async_copy(eye_hbm.at[idx_g.at[i + 1]], buf1, sem1)
            pltpu.make_async_copy(eye_hbm.at[idx_g.at[i]], buf0, sem0).wait()
            pltpu.sync_copy(buf0, acc.at[idx_r.at[i]], add=True)
            pltpu.async_copy(eye_hbm.at[idx_g.at[i + 2]], buf0, sem0)
            pltpu.make_async_copy(eye_hbm.at[idx_g.at[i + 1]], buf1, sem1).wait()
            pltpu.sync_copy(buf1, acc.at[idx_r.at[i + 1]], add=True)

        last = NCHUNK - 1
        pltpu.make_async_copy(eye_hbm.at[idx_g.at[last]], buf0, sem0).wait()
        pltpu.sync_copy(buf0, acc.at[idx_r.at[last]], add=True)

        plsc.subcore_barrier()

        @pl.when(s < NR // 8)
        def _():
            pltpu.sync_copy(acc.at[pl.ds(s * 8, 8)],
                            out_hbm.at[c, pl.ds(s * 8, 8)])

    return deg_kernel(dst_r, eye_h, zeros_h)


def _sc_aggregate(src_r, dst_r, rows, zeros_h, base):
    @functools.partial(
        pl.kernel,
        out_type=jax.ShapeDtypeStruct((NC, NHP, DH), jnp.float32),
        mesh=plsc.VectorSubcoreMesh(**_MESH),
        scratch_types=[
            pltpu.VMEM((NCHUNK, K), jnp.int32),
            pltpu.VMEM((NCHUNK, K), jnp.int32),
            pltpu.VMEM((K, DH), jnp.float32),
            pltpu.VMEM((K, DH), jnp.float32),
            pltpu.VMEM_SHARED((NHP, DH), jnp.float32),
            pltpu.SemaphoreType.DMA,
            pltpu.SemaphoreType.DMA,
        ],
    )
    def agg_kernel(src_hbm, dst_hbm, rows_hbm, z_hbm, out_hbm,
                   idx_s, idx_d, buf0, buf1, acc, sem0, sem1):
        c = lax.axis_index("c")
        s = lax.axis_index("s")
        wid = c * NS + s
        pltpu.sync_copy(z_hbm.at[pl.ds(s * RH, RH)], acc.at[pl.ds(s * RH, RH)])

        @pl.when(s == NS - 1)
        def _():
            pltpu.sync_copy(z_hbm.at[pl.ds(NS * RH, HTAIL)],
                            acc.at[pl.ds(NS * RH, HTAIL)])

        pltpu.sync_copy(src_hbm.at[wid], idx_s)
        pltpu.sync_copy(dst_hbm.at[wid], idx_d)

        @pl.loop(0, NCHUNK)
        def _(i):
            for k in range(K // 16):
                v = idx_d[i, pl.ds(k * 16, 16)]
                keep = jnp.logical_and(v >= base, v < base + NH)
                idx_d[i, pl.ds(k * 16, 16)] = jnp.where(keep, v - base, GB)

        plsc.subcore_barrier()

        pltpu.async_copy(rows_hbm.at[idx_s.at[0]], buf0, sem0)

        @pl.loop(0, NCHUNK - 1, step=2)
        def _(i):
            pltpu.async_copy(rows_hbm.at[idx_s.at[i + 1]], buf1, sem1)
            pltpu.make_async_copy(rows_hbm.at[idx_s.at[i]], buf0, sem0).wait()
            pltpu.sync_copy(buf0, acc.at[idx_d.at[i]], add=True)
            pltpu.async_copy(rows_hbm.at[idx_s.at[i + 2]], buf0, sem0)
            pltpu.make_async_copy(rows_hbm.at[idx_s.at[i + 1]], buf1, sem1).wait()
            pltpu.sync_copy(buf1, acc.at[idx_d.at[i + 1]], add=True)

        last = NCHUNK - 1
        pltpu.make_async_copy(rows_hbm.at[idx_s.at[last]], buf0, sem0).wait()
        pltpu.sync_copy(buf0, acc.at[idx_d.at[last]], add=True)

        plsc.subcore_barrier()
        pltpu.sync_copy(acc.at[pl.ds(s * RH, RH)],
                        out_hbm.at[c, pl.ds(s * RH, RH)])

        @pl.when(s == NS - 1)
        def _():
            pltpu.sync_copy(acc.at[pl.ds(NS * RH, HTAIL)],
                            out_hbm.at[c, pl.ds(NS * RH, HTAIL)])

    return agg_kernel(src_r, dst_r, rows, zeros_h)


def _tc_matmul(x, w):
    def body(x_ref, w_ref, o_ref):
        o_ref[...] = jnp.dot(x_ref[...], w_ref[...],
                             preferred_element_type=jnp.float32)

    return pl.pallas_call(
        body, out_shape=jax.ShapeDtypeStruct((x.shape[0], w.shape[1]),
                                             jnp.float32))(x, w)


def _tc_scale(xw, deg):
    def body(xw_ref, deg_ref, y_ref, dinv_ref):
        dinv = lax.rsqrt(deg_ref[...] + 1.0)
        dinv_ref[...] = dinv
        y_ref[...] = xw_ref[...] * dinv

    return pl.pallas_call(
        body,
        out_shape=[jax.ShapeDtypeStruct((N, DH), jnp.float32),
                   jax.ShapeDtypeStruct((N, 1), jnp.float32)])(xw, deg)


def _tc_mid(pa, pb, y1, dinv, b1):
    def body(pa_ref, pb_ref, y1_ref, dinv_ref, b1_ref, o_ref):
        agg_lo = pa_ref[0, 0:NH] + pa_ref[1, 0:NH]
        agg_hi = pb_ref[0, 0:NH] + pb_ref[1, 0:NH]
        agg = jnp.concatenate([agg_lo, agg_hi], axis=0)
        dv = dinv_ref[...]
        h = jnp.maximum((agg + y1_ref[...]) * dv + b1_ref[...], 0.0)
        o_ref[...] = h * dv

    return pl.pallas_call(
        body, out_shape=jax.ShapeDtypeStruct((N, DH), jnp.float32))(
            pa, pb, y1, dinv, b1)


def _tc_final(pa, pb, m, dinv, w2, b2):
    def body(pa_ref, pb_ref, m_ref, dinv_ref, w2_ref, b2_ref, o_ref):
        agg_lo = pa_ref[0, 0:NH] + pa_ref[1, 0:NH]
        agg_hi = pb_ref[0, 0:NH] + pb_ref[1, 0:NH]
        agg = jnp.concatenate([agg_lo, agg_hi], axis=0) + m_ref[...]
        o_ref[...] = (jnp.dot(agg, w2_ref[...],
                              preferred_element_type=jnp.float32)
                      * dinv_ref[...] + b2_ref[...])

    return pl.pallas_call(
        body, out_shape=jax.ShapeDtypeStruct((N, DO), jnp.float32))(
            pa, pb, m, dinv, w2, b2)


def kernel(x, edge_index, W1, b1, W2, b2):
    src_r = edge_index[0].reshape(NW, NCHUNK, K)
    dst_r = edge_index[1].reshape(NW, NCHUNK, K)
    zeros_h = jnp.zeros((NHP, DH), jnp.float32)
    eye128 = jnp.tile(jnp.eye(DH, dtype=jnp.float32), (NW, 1))
    b1r = b1.reshape(1, DH)
    b2r = b2.reshape(1, DO)

    degp = _sc_degree(dst_r, eye128, zeros_h)
    deg = (degp[0] + degp[1]).reshape(NR * DH)[:N].reshape(N, 1)
    xw = _tc_matmul(x, W1)
    y1, dinv = _tc_scale(xw, deg)
    p1a = _sc_aggregate(src_r, dst_r, y1, zeros_h, 0)
    p1b = _sc_aggregate(src_r, dst_r, y1, zeros_h, NH)
    m = _tc_mid(p1a, p1b, y1, dinv, b1r)
    p2a = _sc_aggregate(src_r, dst_r, m, zeros_h, 0)
    p2b = _sc_aggregate(src_r, dst_r, m, zeros_h, NH)
    return _tc_final(p2a, p2b, m, dinv, W2, b2r)


# speedup vs baseline: 17.3906x; 1.2246x over previous
import functools

import jax
import jax.numpy as jnp
from jax import lax
from jax.experimental import pallas as pl
from jax.experimental.pallas import tpu as pltpu
from jax.experimental.pallas import tpu_sc as plsc

N = 10000
E = 320000
DH = 128
DO = 8
DOP = 16
NC = 2
NS = 16
NW = NC * NS
EW = E // NW
K = 80
NCHUNK = EW // K
RQ = 624
TAIL = N - NS * RQ
NH = N // 2
NHP = NH + 8
GB = NH
RH = 312
HTAIL = NHP - NS * RH

_MESH = dict(core_axis_name="c", subcore_axis_name="s")

import numpy as _np
_PLACE = _np.zeros((8, DO, DH), _np.float32)
for _k in range(8):
    for _j in range(DO):
        _PLACE[_k, _j, 16 * _k + _j] = 1.0


NR = 80


def _sc_degree(dst_r, eye_h, zeros_h):
    @functools.partial(
        pl.kernel,
        out_type=jax.ShapeDtypeStruct((NC, NR, DH), jnp.float32),
        mesh=plsc.VectorSubcoreMesh(**_MESH),
        scratch_types=[
            pltpu.VMEM((NCHUNK, K), jnp.int32),
            pltpu.VMEM((NCHUNK, K), jnp.int32),
            pltpu.VMEM((K, DH), jnp.float32),
            pltpu.VMEM((K, DH), jnp.float32),
            pltpu.VMEM_SHARED((NR, DH), jnp.float32),
            pltpu.SemaphoreType.DMA,
            pltpu.SemaphoreType.DMA,
        ],
    )
    def deg_kernel(dst_hbm, eye_hbm, z_hbm, out_hbm,
                   idx_g, idx_r, buf0, buf1, acc, sem0, sem1):
        c = lax.axis_index("c")
        s = lax.axis_index("s")
        wid = c * NS + s

        @pl.when(s < NR // 8)
        def _():
            pltpu.sync_copy(z_hbm.at[pl.ds(s * 8, 8)], acc.at[pl.ds(s * 8, 8)])

        pltpu.sync_copy(dst_hbm.at[wid], idx_r)

        @pl.loop(0, NCHUNK)
        def _(i):
            for k in range(K // 16):
                v = idx_r[i, pl.ds(k * 16, 16)]
                r = lax.shift_right_logical(v, 7)
                idx_g[i, pl.ds(k * 16, 16)] = (v - r * 128) + wid * DH
                idx_r[i, pl.ds(k * 16, 16)] = r

        plsc.subcore_barrier()

        pltpu.async_copy(eye_hbm.at[idx_g.at[0]], buf0, sem0)

        @pl.loop(0, NCHUNK - 1, step=2)
        def _(i):
            pltpu.async_copy(eye_hbm.at[idx_g.at[i + 1]], buf1, sem1)
            pltpu.make_async_copy(eye_hbm.at[idx_g.at[i]], buf0, sem0).wait()
            pltpu.sync_copy(buf0, acc.at[idx_r.at[i]], add=True)
            pltpu.async_copy(eye_hbm.at[idx_g.at[i + 2]], buf0, sem0)
            pltpu.make_async_copy(eye_hbm.at[idx_g.at[i + 1]], buf1, sem1).wait()
            pltpu.sync_copy(buf1, acc.at[idx_r.at[i + 1]], add=True)

        last = NCHUNK - 1
        pltpu.make_async_copy(eye_hbm.at[idx_g.at[last]], buf0, sem0).wait()
        pltpu.sync_copy(buf0, acc.at[idx_r.at[last]], add=True)

        plsc.subcore_barrier()

        @pl.when(s < NR // 8)
        def _():
            pltpu.sync_copy(acc.at[pl.ds(s * 8, 8)],
                            out_hbm.at[c, pl.ds(s * 8, 8)])

    return deg_kernel(dst_r, eye_h, zeros_h)


def _sc_aggregate(src_r, dst_r, rows, zeros_h, base):
    @functools.partial(
        pl.kernel,
        out_type=jax.ShapeDtypeStruct((NC, NHP, DH), jnp.float32),
        mesh=plsc.VectorSubcoreMesh(**_MESH),
        scratch_types=[
            pltpu.VMEM((NCHUNK, K), jnp.int32),
            pltpu.VMEM((NCHUNK, K), jnp.int32),
            pltpu.VMEM((K, DH), jnp.float32),
            pltpu.VMEM((K, DH), jnp.float32),
            pltpu.VMEM_SHARED((NHP, DH), jnp.float32),
            pltpu.SemaphoreType.DMA,
            pltpu.SemaphoreType.DMA,
        ],
    )
    def agg_kernel(src_hbm, dst_hbm, rows_hbm, z_hbm, out_hbm,
                   idx_s, idx_d, buf0, buf1, acc, sem0, sem1):
        c = lax.axis_index("c")
        s = lax.axis_index("s")
        wid = c * NS + s
        pltpu.sync_copy(z_hbm.at[pl.ds(s * RH, RH)], acc.at[pl.ds(s * RH, RH)])

        @pl.when(s == NS - 1)
        def _():
            pltpu.sync_copy(z_hbm.at[pl.ds(NS * RH, HTAIL)],
                            acc.at[pl.ds(NS * RH, HTAIL)])

        pltpu.sync_copy(src_hbm.at[wid], idx_s)
        pltpu.sync_copy(dst_hbm.at[wid], idx_d)

        @pl.loop(0, NCHUNK)
        def _(i):
            for k in range(K // 16):
                v = idx_d[i, pl.ds(k * 16, 16)]
                keep = jnp.logical_and(v >= base, v < base + NH)
                idx_d[i, pl.ds(k * 16, 16)] = jnp.where(keep, v - base, GB)

        plsc.subcore_barrier()

        pltpu.async_copy(rows_hbm.at[idx_s.at[0]], buf0, sem0)

        @pl.loop(0, NCHUNK - 1, step=2)
        def _(i):
            pltpu.async_copy(rows_hbm.at[idx_s.at[i + 1]], buf1, sem1)
            pltpu.make_async_copy(rows_hbm.at[idx_s.at[i]], buf0, sem0).wait()
            pltpu.sync_copy(buf0, acc.at[idx_d.at[i]], add=True)
            pltpu.async_copy(rows_hbm.at[idx_s.at[i + 2]], buf0, sem0)
            pltpu.make_async_copy(rows_hbm.at[idx_s.at[i + 1]], buf1, sem1).wait()
            pltpu.sync_copy(buf1, acc.at[idx_d.at[i + 1]], add=True)

        last = NCHUNK - 1
        pltpu.make_async_copy(rows_hbm.at[idx_s.at[last]], buf0, sem0).wait()
        pltpu.sync_copy(buf0, acc.at[idx_d.at[last]], add=True)

        plsc.subcore_barrier()
        pltpu.sync_copy(acc.at[pl.ds(s * RH, RH)],
                        out_hbm.at[c, pl.ds(s * RH, RH)])

        @pl.when(s == NS - 1)
        def _():
            pltpu.sync_copy(acc.at[pl.ds(NS * RH, HTAIL)],
                            out_hbm.at[c, pl.ds(NS * RH, HTAIL)])

    return agg_kernel(src_r, dst_r, rows, zeros_h)


NR2 = 1256


def _sc_aggregate8(src_r, dst_r, table, zeros_h):
    @functools.partial(
        pl.kernel,
        out_type=jax.ShapeDtypeStruct((NC, NR2, DH), jnp.float32),
        mesh=plsc.VectorSubcoreMesh(**_MESH),
        scratch_types=[
            pltpu.VMEM((NCHUNK, K), jnp.int32),
            pltpu.VMEM((NCHUNK, K), jnp.int32),
            pltpu.VMEM((K, DH), jnp.float32),
            pltpu.VMEM((K, DH), jnp.float32),
            pltpu.VMEM_SHARED((NR2, DH), jnp.float32),
            pltpu.SemaphoreType.DMA,
            pltpu.SemaphoreType.DMA,
        ],
    )
    def agg8_kernel(src_hbm, dst_hbm, tab_hbm, z_hbm, out_hbm,
                    idx_s, idx_d, buf0, buf1, acc, sem0, sem1):
        c = lax.axis_index("c")
        s = lax.axis_index("s")
        wid = c * NS + s

        @pl.when(s < NS - 1)
        def _():
            pltpu.sync_copy(z_hbm.at[pl.ds(s * 80, 80)],
                            acc.at[pl.ds(s * 80, 80)])

        @pl.when(s == NS - 1)
        def _():
            pltpu.sync_copy(z_hbm.at[pl.ds(1200, NR2 - 1200)],
                            acc.at[pl.ds(1200, NR2 - 1200)])

        pltpu.sync_copy(src_hbm.at[wid], idx_s)
        pltpu.sync_copy(dst_hbm.at[wid], idx_d)

        @pl.loop(0, NCHUNK)
        def _(i):
            for k in range(K // 16):
                v = idx_d[i, pl.ds(k * 16, 16)]
                r = lax.shift_right_logical(v, 3)
                idx_s[i, pl.ds(k * 16, 16)] = (
                    (v - r * 8) * N + idx_s[i, pl.ds(k * 16, 16)])
                idx_d[i, pl.ds(k * 16, 16)] = r

        plsc.subcore_barrier()

        pltpu.async_copy(tab_hbm.at[idx_s.at[0]], buf0, sem0)

        @pl.loop(0, NCHUNK - 1, step=2)
        def _(i):
            pltpu.async_copy(tab_hbm.at[idx_s.at[i + 1]], buf1, sem1)
            pltpu.make_async_copy(tab_hbm.at[idx_s.at[i]], buf0, sem0).wait()
            pltpu.sync_copy(buf0, acc.at[idx_d.at[i]], add=True)
            pltpu.async_copy(tab_hbm.at[idx_s.at[i + 2]], buf0, sem0)
            pltpu.make_async_copy(tab_hbm.at[idx_s.at[i + 1]], buf1, sem1).wait()
            pltpu.sync_copy(buf1, acc.at[idx_d.at[i + 1]], add=True)

        last = NCHUNK - 1
        pltpu.make_async_copy(tab_hbm.at[idx_s.at[last]], buf0, sem0).wait()
        pltpu.sync_copy(buf0, acc.at[idx_d.at[last]], add=True)

        plsc.subcore_barrier()

        @pl.when(s < NS - 1)
        def _():
            pltpu.sync_copy(acc.at[pl.ds(s * 80, 80)],
                            out_hbm.at[c, pl.ds(s * 80, 80)])

        @pl.when(s == NS - 1)
        def _():
            pltpu.sync_copy(acc.at[pl.ds(1200, NR2 - 1200)],
                            out_hbm.at[c, pl.ds(1200, NR2 - 1200)])

    return agg8_kernel(src_r, dst_r, table, zeros_h)


def _tc_build_table(m, w2, p):
    def body(m_ref, w2_ref, p_ref, o_ref):
        y2 = jnp.dot(m_ref[...], w2_ref[...],
                     preferred_element_type=jnp.float32)
        o_ref[0, :, :] = jnp.dot(y2, p_ref[0],
                                 preferred_element_type=jnp.float32)

    return pl.pallas_call(
        body,
        grid=(8,),
        in_specs=[pl.BlockSpec((N, DH), lambda k: (0, 0)),
                  pl.BlockSpec((DH, DO), lambda k: (0, 0)),
                  pl.BlockSpec((1, DO, DH), lambda k: (k, 0, 0))],
        out_specs=pl.BlockSpec((1, N, DH), lambda k: (k, 0, 0)),
        out_shape=jax.ShapeDtypeStruct((8, N, DH), jnp.float32),
    )(m, w2, p)


def _tc_matmul(x, w):
    def body(x_ref, w_ref, o_ref):
        o_ref[...] = jnp.dot(x_ref[...], w_ref[...],
                             preferred_element_type=jnp.float32)

    return pl.pallas_call(
        body, out_shape=jax.ShapeDtypeStruct((x.shape[0], w.shape[1]),
                                             jnp.float32))(x, w)


def _tc_scale(xw, deg):
    def body(xw_ref, deg_ref, y_ref, dinv_ref):
        dinv = lax.rsqrt(deg_ref[...] + 1.0)
        dinv_ref[...] = dinv
        y_ref[...] = xw_ref[...] * dinv

    return pl.pallas_call(
        body,
        out_shape=[jax.ShapeDtypeStruct((N, DH), jnp.float32),
                   jax.ShapeDtypeStruct((N, 1), jnp.float32)])(xw, deg)


def _tc_mid(pa, pb, y1, dinv, b1):
    def body(pa_ref, pb_ref, y1_ref, dinv_ref, b1_ref, o_ref):
        agg_lo = pa_ref[0, 0:NH] + pa_ref[1, 0:NH]
        agg_hi = pb_ref[0, 0:NH] + pb_ref[1, 0:NH]
        agg = jnp.concatenate([agg_lo, agg_hi], axis=0)
        dv = dinv_ref[...]
        h = jnp.maximum((agg + y1_ref[...]) * dv + b1_ref[...], 0.0)
        o_ref[...] = h * dv

    return pl.pallas_call(
        body, out_shape=jax.ShapeDtypeStruct((N, DH), jnp.float32))(
            pa, pb, y1, dinv, b1)


def _tc_final(aggy2, m, dinv, w2, b2):
    def body(agg_ref, m_ref, dinv_ref, w2_ref, b2_ref, o_ref):
        y2 = jnp.dot(m_ref[...], w2_ref[...],
                     preferred_element_type=jnp.float32)
        o_ref[...] = (agg_ref[...] + y2) * dinv_ref[...] + b2_ref[...]

    return pl.pallas_call(
        body, out_shape=jax.ShapeDtypeStruct((N, DO), jnp.float32))(
            aggy2, m, dinv, w2, b2)


def kernel(x, edge_index, W1, b1, W2, b2):
    src_r = edge_index[0].reshape(NW, NCHUNK, K)
    dst_r = edge_index[1].reshape(NW, NCHUNK, K)
    zeros_h = jnp.zeros((NHP, DH), jnp.float32)
    eye128 = jnp.tile(jnp.eye(DH, dtype=jnp.float32), (NW, 1))
    b1r = b1.reshape(1, DH)
    b2r = b2.reshape(1, DO)

    degp = _sc_degree(dst_r, eye128, zeros_h)
    deg = (degp[0] + degp[1]).reshape(NR * DH)[:N].reshape(N, 1)
    xw = _tc_matmul(x, W1)
    y1, dinv = _tc_scale(xw, deg)
    p1a = _sc_aggregate(src_r, dst_r, y1, zeros_h, 0)
    p1b = _sc_aggregate(src_r, dst_r, y1, zeros_h, NH)
    m = _tc_mid(p1a, p1b, y1, dinv, b1r)
    table = _tc_build_table(m, W2, jnp.asarray(_PLACE)).reshape(8 * N, DH)
    aggp = _sc_aggregate8(src_r, dst_r, table, zeros_h)
    aggy2 = (aggp[0] + aggp[1]).reshape(NR2 * 8, 16)[:N, :DO]
    return _tc_final(aggy2, m, dinv, W2, b2r)


# speedup vs baseline: 17.8409x; 1.0259x over previous
import functools

import jax
import jax.numpy as jnp
from jax import lax
from jax.experimental import pallas as pl
from jax.experimental.pallas import tpu as pltpu
from jax.experimental.pallas import tpu_sc as plsc

N = 10000
E = 320000
DH = 128
DO = 8
DOP = 16
NC = 2
NS = 16
NW = NC * NS
EW = E // NW
K = 80
NCHUNK = EW // K
RQ = 624
TAIL = N - NS * RQ
NH = N // 2
NHP = NH + 8
GB = NH
RH = 312
HTAIL = NHP - NS * RH

_MESH = dict(core_axis_name="c", subcore_axis_name="s")

import numpy as _np
_PLACE = _np.zeros((8, DO, DH), _np.float32)
for _k in range(8):
    for _j in range(DO):
        _PLACE[_k, _j, 16 * _k + _j] = 1.0


NR = 80


def _sc_degree(dst_r, eye_h, zeros_h):
    @functools.partial(
        pl.kernel,
        out_type=jax.ShapeDtypeStruct((NC, NR, DH), jnp.float32),
        mesh=plsc.VectorSubcoreMesh(**_MESH),
        scratch_types=[
            pltpu.VMEM((NCHUNK, K), jnp.int32),
            pltpu.VMEM((NCHUNK, K), jnp.int32),
            pltpu.VMEM((K, DH), jnp.float32),
            pltpu.VMEM((K, DH), jnp.float32),
            pltpu.VMEM_SHARED((NR, DH), jnp.float32),
            pltpu.SemaphoreType.DMA,
            pltpu.SemaphoreType.DMA,
        ],
    )
    def deg_kernel(dst_hbm, eye_hbm, z_hbm, out_hbm,
                   idx_g, idx_r, buf0, buf1, acc, sem0, sem1):
        c = lax.axis_index("c")
        s = lax.axis_index("s")
        wid = c * NS + s

        @pl.when(s < NR // 8)
        def _():
            pltpu.sync_copy(z_hbm.at[pl.ds(s * 8, 8)], acc.at[pl.ds(s * 8, 8)])

        pltpu.sync_copy(dst_hbm.at[wid], idx_r)

        @pl.loop(0, NCHUNK)
        def _(i):
            for k in range(K // 16):
                v = idx_r[i, pl.ds(k * 16, 16)]
                r = lax.shift_right_logical(v, 7)
                idx_g[i, pl.ds(k * 16, 16)] = (v - r * 128) + wid * DH
                idx_r[i, pl.ds(k * 16, 16)] = r

        plsc.subcore_barrier()

        pltpu.async_copy(eye_hbm.at[idx_g.at[0]], buf0, sem0)

        @pl.loop(0, NCHUNK - 1, step=2)
        def _(i):
            pltpu.async_copy(eye_hbm.at[idx_g.at[i + 1]], buf1, sem1)
            pltpu.make_async_copy(eye_hbm.at[idx_g.at[i]], buf0, sem0).wait()
            pltpu.sync_copy(buf0, acc.at[idx_r.at[i]], add=True)
            pltpu.async_copy(eye_hbm.at[idx_g.at[i + 2]], buf0, sem0)
            pltpu.make_async_copy(eye_hbm.at[idx_g.at[i + 1]], buf1, sem1).wait()
            pltpu.sync_copy(buf1, acc.at[idx_r.at[i + 1]], add=True)

        last = NCHUNK - 1
        pltpu.make_async_copy(eye_hbm.at[idx_g.at[last]], buf0, sem0).wait()
        pltpu.sync_copy(buf0, acc.at[idx_r.at[last]], add=True)

        plsc.subcore_barrier()

        @pl.when(s < NR // 8)
        def _():
            pltpu.sync_copy(acc.at[pl.ds(s * 8, 8)],
                            out_hbm.at[c, pl.ds(s * 8, 8)])

    return deg_kernel(dst_r, eye_h, zeros_h)


NCH2 = 2 * NCHUNK


def _sc_aggregate(src_r, dst_r, rows, zeros_h):
    @functools.partial(
        pl.kernel,
        out_type=jax.ShapeDtypeStruct((NC, NHP, DH), jnp.float32),
        mesh=plsc.VectorSubcoreMesh(**_MESH),
        scratch_types=[
            pltpu.VMEM((NCH2, K), jnp.int32),
            pltpu.VMEM((NCH2, K), jnp.int32),
            pltpu.VMEM((K, DH), jnp.float32),
            pltpu.VMEM((K, DH), jnp.float32),
            pltpu.VMEM_SHARED((NHP, DH), jnp.float32),
            pltpu.SemaphoreType.DMA,
            pltpu.SemaphoreType.DMA,
        ],
    )
    def agg_kernel(src_hbm, dst_hbm, rows_hbm, z_hbm, out_hbm,
                   idx_s, idx_d, buf0, buf1, acc, sem0, sem1):
        c = lax.axis_index("c")
        s = lax.axis_index("s")
        base = c * NH
        pltpu.sync_copy(z_hbm.at[pl.ds(s * RH, RH)], acc.at[pl.ds(s * RH, RH)])

        @pl.when(s == NS - 1)
        def _():
            pltpu.sync_copy(z_hbm.at[pl.ds(NS * RH, HTAIL)],
                            acc.at[pl.ds(NS * RH, HTAIL)])

        pltpu.sync_copy(src_hbm.at[s], idx_s.at[pl.ds(0, NCHUNK)])
        pltpu.sync_copy(src_hbm.at[s + NS], idx_s.at[pl.ds(NCHUNK, NCHUNK)])
        pltpu.sync_copy(dst_hbm.at[s], idx_d.at[pl.ds(0, NCHUNK)])
        pltpu.sync_copy(dst_hbm.at[s + NS], idx_d.at[pl.ds(NCHUNK, NCHUNK)])

        @pl.loop(0, NCH2)
        def _(i):
            for k in range(K // 16):
                v = idx_d[i, pl.ds(k * 16, 16)]
                keep = jnp.logical_and(v >= base, v < base + NH)
                idx_d[i, pl.ds(k * 16, 16)] = jnp.where(keep, v - base, GB)

        plsc.subcore_barrier()

        pltpu.async_copy(rows_hbm.at[idx_s.at[0]], buf0, sem0)

        @pl.loop(0, NCH2 - 2, step=2)
        def _(i):
            pltpu.async_copy(rows_hbm.at[idx_s.at[i + 1]], buf1, sem1)
            pltpu.make_async_copy(rows_hbm.at[idx_s.at[i]], buf0, sem0).wait()
            pltpu.sync_copy(buf0, acc.at[idx_d.at[i]], add=True)
            pltpu.async_copy(rows_hbm.at[idx_s.at[i + 2]], buf0, sem0)
            pltpu.make_async_copy(rows_hbm.at[idx_s.at[i + 1]], buf1, sem1).wait()
            pltpu.sync_copy(buf1, acc.at[idx_d.at[i + 1]], add=True)

        pltpu.async_copy(rows_hbm.at[idx_s.at[NCH2 - 1]], buf1, sem1)
        pltpu.make_async_copy(rows_hbm.at[idx_s.at[NCH2 - 2]], buf0, sem0).wait()
        pltpu.sync_copy(buf0, acc.at[idx_d.at[NCH2 - 2]], add=True)
        pltpu.make_async_copy(rows_hbm.at[idx_s.at[NCH2 - 1]], buf1, sem1).wait()
        pltpu.sync_copy(buf1, acc.at[idx_d.at[NCH2 - 1]], add=True)

        plsc.subcore_barrier()
        pltpu.sync_copy(acc.at[pl.ds(s * RH, RH)],
                        out_hbm.at[c, pl.ds(s * RH, RH)])

        @pl.when(s == NS - 1)
        def _():
            pltpu.sync_copy(acc.at[pl.ds(NS * RH, HTAIL)],
                            out_hbm.at[c, pl.ds(NS * RH, HTAIL)])

    return agg_kernel(src_r, dst_r, rows, zeros_h)


NR2 = 1256


def _sc_aggregate8(src_r, dst_r, table, zeros_h):
    @functools.partial(
        pl.kernel,
        out_type=jax.ShapeDtypeStruct((NC, NR2, DH), jnp.float32),
        mesh=plsc.VectorSubcoreMesh(**_MESH),
        scratch_types=[
            pltpu.VMEM((NCHUNK, K), jnp.int32),
            pltpu.VMEM((NCHUNK, K), jnp.int32),
            pltpu.VMEM((K, DH), jnp.float32),
            pltpu.VMEM((K, DH), jnp.float32),
            pltpu.VMEM_SHARED((NR2, DH), jnp.float32),
            pltpu.SemaphoreType.DMA,
            pltpu.SemaphoreType.DMA,
        ],
    )
    def agg8_kernel(src_hbm, dst_hbm, tab_hbm, z_hbm, out_hbm,
                    idx_s, idx_d, buf0, buf1, acc, sem0, sem1):
        c = lax.axis_index("c")
        s = lax.axis_index("s")
        wid = c * NS + s

        @pl.when(s < NS - 1)
        def _():
            pltpu.sync_copy(z_hbm.at[pl.ds(s * 80, 80)],
                            acc.at[pl.ds(s * 80, 80)])

        @pl.when(s == NS - 1)
        def _():
            pltpu.sync_copy(z_hbm.at[pl.ds(1200, NR2 - 1200)],
                            acc.at[pl.ds(1200, NR2 - 1200)])

        pltpu.sync_copy(src_hbm.at[wid], idx_s)
        pltpu.sync_copy(dst_hbm.at[wid], idx_d)

        @pl.loop(0, NCHUNK)
        def _(i):
            for k in range(K // 16):
                v = idx_d[i, pl.ds(k * 16, 16)]
                r = lax.shift_right_logical(v, 3)
                idx_s[i, pl.ds(k * 16, 16)] = (
                    (v - r * 8) * N + idx_s[i, pl.ds(k * 16, 16)])
                idx_d[i, pl.ds(k * 16, 16)] = r

        plsc.subcore_barrier()

        pltpu.async_copy(tab_hbm.at[idx_s.at[0]], buf0, sem0)

        @pl.loop(0, NCHUNK - 1, step=2)
        def _(i):
            pltpu.async_copy(tab_hbm.at[idx_s.at[i + 1]], buf1, sem1)
            pltpu.make_async_copy(tab_hbm.at[idx_s.at[i]], buf0, sem0).wait()
            pltpu.sync_copy(buf0, acc.at[idx_d.at[i]], add=True)
            pltpu.async_copy(tab_hbm.at[idx_s.at[i + 2]], buf0, sem0)
            pltpu.make_async_copy(tab_hbm.at[idx_s.at[i + 1]], buf1, sem1).wait()
            pltpu.sync_copy(buf1, acc.at[idx_d.at[i + 1]], add=True)

        last = NCHUNK - 1
        pltpu.make_async_copy(tab_hbm.at[idx_s.at[last]], buf0, sem0).wait()
        pltpu.sync_copy(buf0, acc.at[idx_d.at[last]], add=True)

        plsc.subcore_barrier()

        @pl.when(s < NS - 1)
        def _():
            pltpu.sync_copy(acc.at[pl.ds(s * 80, 80)],
                            out_hbm.at[c, pl.ds(s * 80, 80)])

        @pl.when(s == NS - 1)
        def _():
            pltpu.sync_copy(acc.at[pl.ds(1200, NR2 - 1200)],
                            out_hbm.at[c, pl.ds(1200, NR2 - 1200)])

    return agg8_kernel(src_r, dst_r, table, zeros_h)


def _tc_build_table(m, w2, p):
    def body(m_ref, w2_ref, p_ref, o_ref):
        y2 = jnp.dot(m_ref[...], w2_ref[...],
                     preferred_element_type=jnp.float32)
        o_ref[0, :, :] = jnp.dot(y2, p_ref[0],
                                 preferred_element_type=jnp.float32)

    return pl.pallas_call(
        body,
        grid=(8,),
        in_specs=[pl.BlockSpec((N, DH), lambda k: (0, 0)),
                  pl.BlockSpec((DH, DO), lambda k: (0, 0)),
                  pl.BlockSpec((1, DO, DH), lambda k: (k, 0, 0))],
        out_specs=pl.BlockSpec((1, N, DH), lambda k: (k, 0, 0)),
        out_shape=jax.ShapeDtypeStruct((8, N, DH), jnp.float32),
    )(m, w2, p)


def _tc_matmul(x, w):
    def body(x_ref, w_ref, o_ref):
        o_ref[...] = jnp.dot(x_ref[...], w_ref[...],
                             preferred_element_type=jnp.float32)

    return pl.pallas_call(
        body, out_shape=jax.ShapeDtypeStruct((x.shape[0], w.shape[1]),
                                             jnp.float32))(x, w)


def _tc_scale(xw, deg):
    def body(xw_ref, deg_ref, y_ref, dinv_ref):
        dinv = lax.rsqrt(deg_ref[...] + 1.0)
        dinv_ref[...] = dinv
        y_ref[...] = xw_ref[...] * dinv

    return pl.pallas_call(
        body,
        out_shape=[jax.ShapeDtypeStruct((N, DH), jnp.float32),
                   jax.ShapeDtypeStruct((N, 1), jnp.float32)])(xw, deg)


def _tc_mid(p1, y1, dinv, b1):
    def body(p1_ref, y1_ref, dinv_ref, b1_ref, o_ref):
        agg = jnp.concatenate([p1_ref[0, 0:NH], p1_ref[1, 0:NH]], axis=0)
        dv = dinv_ref[...]
        h = jnp.maximum((agg + y1_ref[...]) * dv + b1_ref[...], 0.0)
        o_ref[...] = h * dv

    return pl.pallas_call(
        body, out_shape=jax.ShapeDtypeStruct((N, DH), jnp.float32))(
            p1, y1, dinv, b1)


def _tc_final(aggy2, m, dinv, w2, b2):
    def body(agg_ref, m_ref, dinv_ref, w2_ref, b2_ref, o_ref):
        y2 = jnp.dot(m_ref[...], w2_ref[...],
                     preferred_element_type=jnp.float32)
        o_ref[...] = (agg_ref[...] + y2) * dinv_ref[...] + b2_ref[...]

    return pl.pallas_call(
        body, out_shape=jax.ShapeDtypeStruct((N, DO), jnp.float32))(
            aggy2, m, dinv, w2, b2)


def kernel(x, edge_index, W1, b1, W2, b2):
    src_r = edge_index[0].reshape(NW, NCHUNK, K)
    dst_r = edge_index[1].reshape(NW, NCHUNK, K)
    zeros_h = jnp.zeros((NHP, DH), jnp.float32)
    eye128 = jnp.tile(jnp.eye(DH, dtype=jnp.float32), (NW, 1))
    b1r = b1.reshape(1, DH)
    b2r = b2.reshape(1, DO)

    degp = _sc_degree(dst_r, eye128, zeros_h)
    deg = (degp[0] + degp[1]).reshape(NR * DH)[:N].reshape(N, 1)
    xw = _tc_matmul(x, W1)
    y1, dinv = _tc_scale(xw, deg)
    p1 = _sc_aggregate(src_r, dst_r, y1, zeros_h)
    m = _tc_mid(p1, y1, dinv, b1r)
    table = _tc_build_table(m, W2, jnp.asarray(_PLACE)).reshape(8 * N, DH)
    aggp = _sc_aggregate8(src_r, dst_r, table, zeros_h)
    aggy2 = (aggp[0] + aggp[1]).reshape(NR2 * 8, 16)[:N, :DO]
    return _tc_final(aggy2, m, dinv, W2, b2r)
